# baseline (speedup 1.0000x reference)
import functools

import jax
import jax.numpy as jnp
from jax import lax
from jax.experimental import pallas as pl
from jax.experimental.pallas import tpu as pltpu
from jax.experimental.pallas import tpu_sc as plsc

N = 10000
E = 320000
H = 16
NC = 2
NS = 16
NW = NC * NS
EPW = E // NW
CH = 125
NCH = EPW // CH
NP = 10240
NPW = NP // NS

_mesh = plsc.VectorSubcoreMesh(core_axis_name="c", subcore_axis_name="s")


def _take16(v, ix):
    return lax.gather(
        v, ix[:, None],
        dimension_numbers=lax.GatherDimensionNumbers(
            offset_dims=(), collapsed_slice_dims=(0,), start_index_map=(0,)),
        slice_sizes=(1,),
        mode=lax.GatherScatterMode.PROMISE_IN_BOUNDS)


@functools.partial(
    pl.kernel,
    out_type=jax.ShapeDtypeStruct((NW * N,), jnp.float32),
    mesh=_mesh,
    scratch_types=[
        pltpu.VMEM((N,), jnp.float32),
        pltpu.VMEM((EPW,), jnp.int32),
    ],
    compiler_params=pltpu.CompilerParams(needs_layout_passes=False,
                                         use_tc_tiling_on_sc=False),
)
def _deg_kernel(dst_hbm, zn_hbm, out_hbm, deg_v, idx_v):
    c = lax.axis_index("c")
    s = lax.axis_index("s")
    wid = s * NC + c
    pltpu.sync_copy(zn_hbm, deg_v)
    pltpu.sync_copy(dst_hbm.at[pl.ds(wid * EPW, EPW)], idx_v)
    lanes = lax.iota(jnp.int32, 16)
    prev_ix = jnp.maximum(lanes - 1, 0)
    next_ix = jnp.minimum(lanes + 1, 15)

    def body(i, _):
        idx = idx_v[pl.ds(i * 16, 16)]
        sk = jnp.sort(idx)
        prev = _take16(sk, prev_ix)
        nxt = _take16(sk, next_ix)
        start = (sk != prev) | (lanes == 0)
        islast = (sk != nxt) | (lanes == 15)
        runstart = plsc.cummax(jnp.where(start, lanes, 0))
        cnt = (lanes - runstart + 1).astype(jnp.float32)
        plsc.addupdate_scatter(deg_v, [sk], cnt, mask=islast)
        return _

    lax.fori_loop(0, EPW // 16, body, None)
    pltpu.sync_copy(deg_v, out_hbm.at[pl.ds(wid * N, N)])


@functools.partial(
    pl.kernel,
    out_type=jax.ShapeDtypeStruct((NC, NP, H), jnp.float32),
    mesh=_mesh,
    scratch_types=[
        pltpu.VMEM_SHARED((NP, H), jnp.float32),
        pltpu.VMEM((NCH, CH), jnp.int32),
        pltpu.VMEM((NCH, CH), jnp.int32),
        pltpu.VMEM((2, CH, H), jnp.float32),
        pltpu.SemaphoreType.DMA,
        pltpu.SemaphoreType.DMA,
    ],
    compiler_params=pltpu.CompilerParams(use_tc_tiling_on_sc=False),
)
def _seg_kernel(hws_hbm, src2_hbm, dst2_hbm, znh_hbm, out_hbm,
                acc_sh, sidx_v, didx_v, rows_v, sem0, sem1):
    c = lax.axis_index("c")
    s = lax.axis_index("s")
    wid = s * NC + c

    pltpu.sync_copy(src2_hbm.at[pl.ds(wid * NCH, NCH)], sidx_v)
    pltpu.sync_copy(dst2_hbm.at[pl.ds(wid * NCH, NCH)], didx_v)
    pltpu.sync_copy(znh_hbm.at[pl.ds(s * NPW, NPW)],
                    acc_sh.at[pl.ds(s * NPW, NPW)])

    def _gather(j, buf, sem):
        return pltpu.async_copy(hws_hbm.at[sidx_v.at[j]], rows_v.at[buf], sem)

    def _gwait(j, buf, sem):
        pltpu.make_async_copy(hws_hbm.at[sidx_v.at[j]], rows_v.at[buf], sem).wait()

    def _scatter(j, buf):
        pltpu.sync_copy(rows_v.at[buf], acc_sh.at[didx_v.at[j]], add=True)

    _gather(0, 0, sem0)
    plsc.subcore_barrier()

    def body(k, _):
        j0 = 2 * k
        j1 = 2 * k + 1
        _gather(j1, 1, sem1)
        _gwait(j0, 0, sem0)
        _scatter(j0, 0)
        _gather(j1 + 1, 0, sem0)
        _gwait(j1, 1, sem1)
        _scatter(j1, 1)
        return _

    lax.fori_loop(0, NCH // 2 - 1, body, None)
    _gather(NCH - 1, 1, sem1)
    _gwait(NCH - 2, 0, sem0)
    _scatter(NCH - 2, 0)
    _gwait(NCH - 1, 1, sem1)
    _scatter(NCH - 1, 1)
    plsc.subcore_barrier()
    pltpu.sync_copy(acc_sh.at[pl.ds(s * NPW, NPW)],
                    out_hbm.at[c].at[pl.ds(s * NPW, NPW)])


@functools.partial(
    pl.kernel,
    out_type=jax.ShapeDtypeStruct((E,), jnp.float32),
    mesh=_mesh,
    scratch_types=[
        pltpu.VMEM((N,), jnp.float32),
        pltpu.VMEM((N,), jnp.float32),
        pltpu.VMEM((EPW,), jnp.int32),
        pltpu.VMEM((EPW,), jnp.int32),
        pltpu.VMEM((EPW,), jnp.float32),
        pltpu.VMEM((EPW,), jnp.float32),
    ],
    compiler_params=pltpu.CompilerParams(needs_layout_passes=False,
                                         use_tc_tiling_on_sc=False),
)
def _score_kernel(a_hbm, b_hbm, cc_hbm, sl_hbm, dl_hbm, out_hbm,
                  a_v, b_v, sl_v, dl_v, c_v, o_v):
    c = lax.axis_index("c")
    s = lax.axis_index("s")
    wid = s * NC + c
    base = wid * EPW
    pltpu.sync_copy(a_hbm, a_v)
    pltpu.sync_copy(b_hbm, b_v)
    pltpu.sync_copy(sl_hbm.at[pl.ds(base, EPW)], sl_v)
    pltpu.sync_copy(dl_hbm.at[pl.ds(base, EPW)], dl_v)
    pltpu.sync_copy(cc_hbm.at[pl.ds(base, EPW)], c_v)

    def body(i, _):
        o = i * 16
        va = plsc.load_gather(a_v, [sl_v[pl.ds(o, 16)]])
        vb = plsc.load_gather(b_v, [dl_v[pl.ds(o, 16)]])
        o_v[pl.ds(o, 16)] = va + vb + c_v[pl.ds(o, 16)]
        return _

    lax.fori_loop(0, EPW // 16, body, None)
    pltpu.sync_copy(o_v, out_hbm.at[pl.ds(base, EPW)])


def _lrelu(v):
    return jnp.where(v >= 0, v, 0.01 * v)


def _dot(a, b):
    return jax.lax.dot(a, b, precision=jax.lax.Precision.HIGHEST,
                       preferred_element_type=jnp.float32)


def _tc1_body(x_ref, w1_ref, b1_ref, w2_ref, b2_ref, wc1_ref, degp_ref,
              hws1_ref, dis_ref):
    h = _lrelu(_dot(x_ref[...], w1_ref[...]) + b1_ref[...][None, :])
    h = _lrelu(_dot(h, w2_ref[...]) + b2_ref[...][None, :])
    deg = jnp.sum(degp_ref[...], axis=0) + 1.0
    dis = lax.rsqrt(deg)
    hws1_ref[pl.ds(0, N), :] = _dot(h, wc1_ref[...]) * dis[:, None]
    dis_ref[pl.ds(0, N)] = dis


_tc1 = pl.pallas_call(
    _tc1_body,
    out_shape=(jax.ShapeDtypeStruct((NP, H), jnp.float32),
               jax.ShapeDtypeStruct((NP,), jnp.float32)),
)


def _tc2_body(t1p_ref, hws1_ref, dis_ref, wc2_ref, bc1_ref, hws2_ref):
    dis = dis_ref[...]
    agg = (t1p_ref[0] + t1p_ref[1] + hws1_ref[...]) * dis[:, None] \
        + bc1_ref[...][None, :]
    h1 = _lrelu(agg)
    hws2_ref[...] = _dot(h1, wc2_ref[...]) * dis[:, None]


_tc2 = pl.pallas_call(
    _tc2_body,
    out_shape=jax.ShapeDtypeStruct((NP, H), jnp.float32),
)


def _tc3_body(t2p_ref, hws2_ref, dis_ref, bc2_ref, wpa_ref, wpb_ref,
              h2_ref, a_ref, bv_ref):
    dis = dis_ref[...]
    agg = (t2p_ref[0] + t2p_ref[1] + hws2_ref[...]) * dis[:, None] \
        + bc2_ref[...][None, :]
    h2 = _lrelu(agg)
    h2_ref[...] = h2[:N, :]
    a_ref[...] = jnp.sum(h2 * wpa_ref[...][None, :], axis=1)[:N]
    bv_ref[...] = jnp.sum(h2 * wpb_ref[...][None, :], axis=1)[:N]


_tc3 = pl.pallas_call(
    _tc3_body,
    out_shape=(jax.ShapeDtypeStruct((N, H), jnp.float32),
               jax.ShapeDtypeStruct((N,), jnp.float32),
               jax.ShapeDtypeStruct((N,), jnp.float32)),
)


def _tc4_body(ea_ref, wpe_ref, bp_ref, c_ref):
    i = pl.program_id(0)
    c_ref[pl.ds(i * _EB, _EB)] = \
        jnp.sum(ea_ref[...] * wpe_ref[...][None, :], axis=1) + bp_ref[0]


_EB = 6400

_tc4 = pl.pallas_call(
    _tc4_body,
    grid=(E // _EB,),
    in_specs=[
        pl.BlockSpec((_EB, H), lambda i: (i, 0)),
        pl.BlockSpec((H,), lambda i: (0,)),
        pl.BlockSpec((1,), lambda i: (0,)),
    ],
    out_specs=pl.BlockSpec((E,), lambda i: (0,)),
    out_shape=jax.ShapeDtypeStruct((E,), jnp.float32),
)


def kernel(x, edge_index, edge_label_index, edge_attr,
           W1, b1, W2, b2, Wc1, bc1, Wc2, bc2, Wp, bp):
    src = edge_index[0]
    dst = edge_index[1]
    src2 = src.reshape(E // CH, CH)
    dst2 = dst.reshape(E // CH, CH)
    zn = jnp.zeros((N,), jnp.float32)
    znh = jnp.zeros((NP, H), jnp.float32)
    wp_flat = Wp.reshape(48)
    wpa, wpb, wpe = wp_flat[0:16], wp_flat[16:32], wp_flat[32:48]

    degp = _deg_kernel(dst, zn).reshape(NW, N)
    hws1, dis = _tc1(x, W1, b1, W2, b2, Wc1, degp)
    t1p = _seg_kernel(hws1, src2, dst2, znh)
    hws2 = _tc2(t1p, hws1, dis, Wc2, bc1)
    t2p = _seg_kernel(hws2, src2, dst2, znh)
    h2, a, bv = _tc3(t2p, hws2, dis, bc2, wpa, wpb)
    cvec = _tc4(edge_attr, wpe, bp)
    out = _score_kernel(a, bv, cvec, edge_label_index[0], edge_label_index[1])
    return (out, h2)


# speedup vs baseline: 1.8162x; 1.8162x over previous
import functools

import jax
import jax.numpy as jnp
from jax import lax
from jax.experimental import pallas as pl
from jax.experimental.pallas import tpu as pltpu
from jax.experimental.pallas import tpu_sc as plsc

N = 10000
E = 320000
H = 16
NC = 2
NS = 16
NW = NC * NS
EPW = E // NW
CH = 125
NCH = EPW // CH
NP = 10240
NPW = NP // NS

_mesh = plsc.VectorSubcoreMesh(core_axis_name="c", subcore_axis_name="s")


def _take16(v, ix):
    return lax.gather(
        v, ix[:, None],
        dimension_numbers=lax.GatherDimensionNumbers(
            offset_dims=(), collapsed_slice_dims=(0,), start_index_map=(0,)),
        slice_sizes=(1,),
        mode=lax.GatherScatterMode.PROMISE_IN_BOUNDS)


@functools.partial(
    pl.kernel,
    out_type=jax.ShapeDtypeStruct((NW * N,), jnp.float32),
    mesh=_mesh,
    scratch_types=[
        pltpu.VMEM((N,), jnp.float32),
        pltpu.VMEM((EPW,), jnp.int32),
    ],
    compiler_params=pltpu.CompilerParams(needs_layout_passes=False,
                                         use_tc_tiling_on_sc=False),
)
def _deg_kernel(dst_hbm, zn_hbm, out_hbm, deg_v, idx_v):
    c = lax.axis_index("c")
    s = lax.axis_index("s")
    wid = s * NC + c
    pltpu.sync_copy(zn_hbm, deg_v)
    pltpu.sync_copy(dst_hbm.at[pl.ds(wid * EPW, EPW)], idx_v)
    lanes = lax.iota(jnp.int32, 16)
    prev_ix = jnp.maximum(lanes - 1, 0)
    next_ix = jnp.minimum(lanes + 1, 15)

    def body(i, _):
        idx = idx_v[pl.ds(i * 16, 16)]
        sk = jnp.sort(idx)
        prev = _take16(sk, prev_ix)
        nxt = _take16(sk, next_ix)
        start = (sk != prev) | (lanes == 0)
        islast = (sk != nxt) | (lanes == 15)
        runstart = plsc.cummax(jnp.where(start, lanes, 0))
        cnt = (lanes - runstart + 1).astype(jnp.float32)
        plsc.addupdate_scatter(deg_v, [sk], cnt, mask=islast)
        return _

    lax.fori_loop(0, EPW // 16, body, None)
    pltpu.sync_copy(deg_v, out_hbm.at[pl.ds(wid * N, N)])


@functools.partial(
    pl.kernel,
    out_type=jax.ShapeDtypeStruct((NC, NP, H), jnp.float32),
    mesh=_mesh,
    scratch_types=[
        pltpu.VMEM_SHARED((NP, H), jnp.float32),
        pltpu.VMEM((NCH, CH), jnp.int32),
        pltpu.VMEM((NCH, CH), jnp.int32),
        pltpu.VMEM((2, CH, H), jnp.float32),
        pltpu.SemaphoreType.DMA,
        pltpu.SemaphoreType.DMA,
    ],
    compiler_params=pltpu.CompilerParams(use_tc_tiling_on_sc=False),
)
def _seg_kernel(hws_hbm, src2_hbm, dst2_hbm, znh_hbm, out_hbm,
                acc_sh, sidx_v, didx_v, rows_v, sem0, sem1):
    c = lax.axis_index("c")
    s = lax.axis_index("s")
    wid = s * NC + c

    pltpu.sync_copy(src2_hbm.at[pl.ds(wid * NCH, NCH)], sidx_v)
    pltpu.sync_copy(dst2_hbm.at[pl.ds(wid * NCH, NCH)], didx_v)
    pltpu.sync_copy(znh_hbm.at[pl.ds(s * NPW, NPW)],
                    acc_sh.at[pl.ds(s * NPW, NPW)])

    def _gather(j, buf, sem):
        return pltpu.async_copy(hws_hbm.at[sidx_v.at[j]], rows_v.at[buf], sem)

    def _gwait(j, buf, sem):
        pltpu.make_async_copy(hws_hbm.at[sidx_v.at[j]], rows_v.at[buf], sem).wait()

    def _scatter(j, buf):
        pltpu.sync_copy(rows_v.at[buf], acc_sh.at[didx_v.at[j]], add=True)

    _gather(0, 0, sem0)
    plsc.subcore_barrier()

    def body(k, _):
        j0 = 2 * k
        j1 = 2 * k + 1
        _gather(j1, 1, sem1)
        _gwait(j0, 0, sem0)
        _scatter(j0, 0)
        _gather(j1 + 1, 0, sem0)
        _gwait(j1, 1, sem1)
        _scatter(j1, 1)
        return _

    lax.fori_loop(0, NCH // 2 - 1, body, None)
    _gather(NCH - 1, 1, sem1)
    _gwait(NCH - 2, 0, sem0)
    _scatter(NCH - 2, 0)
    _gwait(NCH - 1, 1, sem1)
    _scatter(NCH - 1, 1)
    plsc.subcore_barrier()
    pltpu.sync_copy(acc_sh.at[pl.ds(s * NPW, NPW)],
                    out_hbm.at[c].at[pl.ds(s * NPW, NPW)])


@functools.partial(
    pl.kernel,
    out_type=jax.ShapeDtypeStruct((E,), jnp.float32),
    mesh=_mesh,
    scratch_types=[
        pltpu.VMEM((N,), jnp.float32),
        pltpu.VMEM((N,), jnp.float32),
        pltpu.VMEM((EPW,), jnp.int32),
        pltpu.VMEM((EPW,), jnp.int32),
        pltpu.VMEM((EPW,), jnp.float32),
        pltpu.VMEM((EPW,), jnp.float32),
    ],
    compiler_params=pltpu.CompilerParams(needs_layout_passes=False,
                                         use_tc_tiling_on_sc=False),
)
def _score_kernel(a_hbm, b_hbm, cc_hbm, sl_hbm, dl_hbm, out_hbm,
                  a_v, b_v, sl_v, dl_v, c_v, o_v):
    c = lax.axis_index("c")
    s = lax.axis_index("s")
    wid = s * NC + c
    base = wid * EPW
    pltpu.sync_copy(a_hbm, a_v)
    pltpu.sync_copy(b_hbm, b_v)
    pltpu.sync_copy(sl_hbm.at[pl.ds(base, EPW)], sl_v)
    pltpu.sync_copy(dl_hbm.at[pl.ds(base, EPW)], dl_v)
    pltpu.sync_copy(cc_hbm.at[pl.ds(base, EPW)], c_v)

    def body(i, _):
        o = i * 16
        va = plsc.load_gather(a_v, [sl_v[pl.ds(o, 16)]])
        vb = plsc.load_gather(b_v, [dl_v[pl.ds(o, 16)]])
        o_v[pl.ds(o, 16)] = va + vb + c_v[pl.ds(o, 16)]
        return _

    lax.fori_loop(0, EPW // 16, body, None)
    pltpu.sync_copy(o_v, out_hbm.at[pl.ds(base, EPW)])


def _lrelu(v):
    return jnp.where(v >= 0, v, 0.01 * v)


def _dot(a, b):
    return jax.lax.dot(a, b, precision=jax.lax.Precision.HIGHEST,
                       preferred_element_type=jnp.float32)


def _tc1_body(x_ref, w1_ref, b1_ref, w2_ref, b2_ref, wc1_ref, degp_ref,
              hws1_ref, dis_ref):
    h = _lrelu(_dot(x_ref[...], w1_ref[...]) + b1_ref[...][None, :])
    h = _lrelu(_dot(h, w2_ref[...]) + b2_ref[...][None, :])
    deg = jnp.sum(degp_ref[...], axis=0) + 1.0
    dis = lax.rsqrt(deg)
    hws1_ref[pl.ds(0, N), :] = _dot(h, wc1_ref[...]) * dis[:, None]
    dis_ref[pl.ds(0, N)] = dis


_tc1 = pl.pallas_call(
    _tc1_body,
    out_shape=(jax.ShapeDtypeStruct((NP, H), jnp.float32),
               jax.ShapeDtypeStruct((NP,), jnp.float32)),
)


def _tc2_body(t1p_ref, hws1_ref, dis_ref, wc2_ref, bc1_ref, hws2_ref):
    dis = dis_ref[...]
    agg = (t1p_ref[0] + t1p_ref[1] + hws1_ref[...]) * dis[:, None] \
        + bc1_ref[...][None, :]
    h1 = _lrelu(agg)
    hws2_ref[...] = _dot(h1, wc2_ref[...]) * dis[:, None]


_tc2 = pl.pallas_call(
    _tc2_body,
    out_shape=jax.ShapeDtypeStruct((NP, H), jnp.float32),
)


def _tc3_body(t2p_ref, hws2_ref, dis_ref, bc2_ref, wpa_ref, wpb_ref,
              h2_ref, a_ref, bv_ref):
    dis = dis_ref[...]
    agg = (t2p_ref[0] + t2p_ref[1] + hws2_ref[...]) * dis[:, None] \
        + bc2_ref[...][None, :]
    h2 = _lrelu(agg)
    h2_ref[...] = h2[:N, :]
    a_ref[...] = jnp.sum(h2 * wpa_ref[...][None, :], axis=1)[:N]
    bv_ref[...] = jnp.sum(h2 * wpb_ref[...][None, :], axis=1)[:N]


_tc3 = pl.pallas_call(
    _tc3_body,
    out_shape=(jax.ShapeDtypeStruct((N, H), jnp.float32),
               jax.ShapeDtypeStruct((N,), jnp.float32),
               jax.ShapeDtypeStruct((N,), jnp.float32)),
)


def _tc4_body(ea_ref, wpe_ref, bp_ref, c_ref):
    i = pl.program_id(0)
    c_ref[pl.ds(i * _EB, _EB)] = \
        jnp.sum(ea_ref[...] * wpe_ref[...][:, None], axis=0) + bp_ref[0]


_EB = 32000

_tc4 = pl.pallas_call(
    _tc4_body,
    grid=(E // _EB,),
    in_specs=[
        pl.BlockSpec((H, _EB), lambda i: (0, i)),
        pl.BlockSpec((H,), lambda i: (0,)),
        pl.BlockSpec((1,), lambda i: (0,)),
    ],
    out_specs=pl.BlockSpec((E,), lambda i: (0,)),
    out_shape=jax.ShapeDtypeStruct((E,), jnp.float32),
)


def kernel(x, edge_index, edge_label_index, edge_attr,
           W1, b1, W2, b2, Wc1, bc1, Wc2, bc2, Wp, bp):
    src = edge_index[0]
    dst = edge_index[1]
    src2 = src.reshape(E // CH, CH)
    dst2 = dst.reshape(E // CH, CH)
    zn = jnp.zeros((N,), jnp.float32)
    znh = jnp.zeros((NP, H), jnp.float32)
    wp_flat = Wp.reshape(48)
    wpa, wpb, wpe = wp_flat[0:16], wp_flat[16:32], wp_flat[32:48]

    degp = _deg_kernel(dst, zn).reshape(NW, N)
    hws1, dis = _tc1(x, W1, b1, W2, b2, Wc1, degp)
    t1p = _seg_kernel(hws1, src2, dst2, znh)
    hws2 = _tc2(t1p, hws1, dis, Wc2, bc1)
    t2p = _seg_kernel(hws2, src2, dst2, znh)
    h2, a, bv = _tc3(t2p, hws2, dis, bc2, wpa, wpb)
    cvec = _tc4(edge_attr.T, wpe, bp)
    out = _score_kernel(a, bv, cvec, edge_label_index[0], edge_label_index[1])
    return (out, h2)


# speedup vs baseline: 1.9998x; 1.1011x over previous
import functools

import jax
import jax.numpy as jnp
from jax import lax
from jax.experimental import pallas as pl
from jax.experimental.pallas import tpu as pltpu
from jax.experimental.pallas import tpu_sc as plsc

N = 10000
E = 320000
H = 16
NC = 2
NS = 16
NW = NC * NS
EPW = E // NW
CH = 125
NCH = EPW // CH
NP = 10240
NPW = NP // NS

_mesh = plsc.VectorSubcoreMesh(core_axis_name="c", subcore_axis_name="s")


def _take16(v, ix):
    return lax.gather(
        v, ix[:, None],
        dimension_numbers=lax.GatherDimensionNumbers(
            offset_dims=(), collapsed_slice_dims=(0,), start_index_map=(0,)),
        slice_sizes=(1,),
        mode=lax.GatherScatterMode.PROMISE_IN_BOUNDS)


@functools.partial(
    pl.kernel,
    out_type=jax.ShapeDtypeStruct((NW * N,), jnp.float32),
    mesh=_mesh,
    scratch_types=[
        pltpu.VMEM((N,), jnp.float32),
        pltpu.VMEM((EPW,), jnp.int32),
    ],
    compiler_params=pltpu.CompilerParams(needs_layout_passes=False,
                                         use_tc_tiling_on_sc=False),
)
def _deg_kernel(dst_hbm, zn_hbm, out_hbm, deg_v, idx_v):
    c = lax.axis_index("c")
    s = lax.axis_index("s")
    wid = s * NC + c
    pltpu.sync_copy(zn_hbm, deg_v)
    pltpu.sync_copy(dst_hbm.at[pl.ds(wid * EPW, EPW)], idx_v)
    lanes = lax.iota(jnp.int32, 16)
    prev_ix = jnp.maximum(lanes - 1, 0)
    next_ix = jnp.minimum(lanes + 1, 15)

    def body(i, _):
        idx = idx_v[pl.ds(i * 16, 16)]
        sk = jnp.sort(idx)
        prev = _take16(sk, prev_ix)
        nxt = _take16(sk, next_ix)
        start = (sk != prev) | (lanes == 0)
        islast = (sk != nxt) | (lanes == 15)
        runstart = plsc.cummax(jnp.where(start, lanes, 0))
        cnt = (lanes - runstart + 1).astype(jnp.float32)
        plsc.addupdate_scatter(deg_v, [sk], cnt, mask=islast)
        return _

    lax.fori_loop(0, EPW // 16, body, None)
    pltpu.sync_copy(deg_v, out_hbm.at[pl.ds(wid * N, N)])


@functools.partial(
    pl.kernel,
    out_type=jax.ShapeDtypeStruct((NC, NP, H), jnp.float32),
    mesh=_mesh,
    scratch_types=[
        pltpu.VMEM_SHARED((NP, H), jnp.float32),
        pltpu.VMEM((NCH, CH), jnp.int32),
        pltpu.VMEM((NCH, CH), jnp.int32),
        pltpu.VMEM((2, CH, H), jnp.float32),
        pltpu.SemaphoreType.DMA,
        pltpu.SemaphoreType.DMA,
    ],
    compiler_params=pltpu.CompilerParams(use_tc_tiling_on_sc=False),
)
def _seg_kernel(hws_hbm, src2_hbm, dst2_hbm, znh_hbm, out_hbm,
                acc_sh, sidx_v, didx_v, rows_v, sem0, sem1):
    c = lax.axis_index("c")
    s = lax.axis_index("s")
    wid = s * NC + c

    pltpu.sync_copy(src2_hbm.at[pl.ds(wid * NCH, NCH)], sidx_v)
    pltpu.sync_copy(dst2_hbm.at[pl.ds(wid * NCH, NCH)], didx_v)
    pltpu.sync_copy(znh_hbm.at[pl.ds(s * NPW, NPW)],
                    acc_sh.at[pl.ds(s * NPW, NPW)])

    def _gather(j, buf, sem):
        return pltpu.async_copy(hws_hbm.at[sidx_v.at[j]], rows_v.at[buf], sem)

    def _gwait(j, buf, sem):
        pltpu.make_async_copy(hws_hbm.at[sidx_v.at[j]], rows_v.at[buf], sem).wait()

    def _scatter(j, buf):
        pltpu.sync_copy(rows_v.at[buf], acc_sh.at[didx_v.at[j]], add=True)

    _gather(0, 0, sem0)
    plsc.subcore_barrier()

    def body(k, _):
        j0 = 2 * k
        j1 = 2 * k + 1
        _gather(j1, 1, sem1)
        _gwait(j0, 0, sem0)
        _scatter(j0, 0)
        _gather(j1 + 1, 0, sem0)
        _gwait(j1, 1, sem1)
        _scatter(j1, 1)
        return _

    lax.fori_loop(0, NCH // 2 - 1, body, None)
    _gather(NCH - 1, 1, sem1)
    _gwait(NCH - 2, 0, sem0)
    _scatter(NCH - 2, 0)
    _gwait(NCH - 1, 1, sem1)
    _scatter(NCH - 1, 1)
    plsc.subcore_barrier()
    pltpu.sync_copy(acc_sh.at[pl.ds(s * NPW, NPW)],
                    out_hbm.at[c].at[pl.ds(s * NPW, NPW)])


@functools.partial(
    pl.kernel,
    out_type=jax.ShapeDtypeStruct((E,), jnp.float32),
    mesh=_mesh,
    scratch_types=[
        pltpu.VMEM((N,), jnp.float32),
        pltpu.VMEM((N,), jnp.float32),
        pltpu.VMEM((EPW,), jnp.int32),
        pltpu.VMEM((EPW,), jnp.int32),
        pltpu.VMEM((EPW,), jnp.float32),
        pltpu.VMEM((EPW,), jnp.float32),
    ],
    compiler_params=pltpu.CompilerParams(needs_layout_passes=False,
                                         use_tc_tiling_on_sc=False),
)
def _score_kernel(a_hbm, b_hbm, cc_hbm, sl_hbm, dl_hbm, out_hbm,
                  a_v, b_v, sl_v, dl_v, c_v, o_v):
    c = lax.axis_index("c")
    s = lax.axis_index("s")
    wid = s * NC + c
    base = wid * EPW
    pltpu.sync_copy(a_hbm, a_v)
    pltpu.sync_copy(b_hbm, b_v)
    pltpu.sync_copy(sl_hbm.at[pl.ds(base, EPW)], sl_v)
    pltpu.sync_copy(dl_hbm.at[pl.ds(base, EPW)], dl_v)
    pltpu.sync_copy(cc_hbm.at[pl.ds(base, EPW)], c_v)

    def body(i, _):
        o = i * 16
        va = plsc.load_gather(a_v, [sl_v[pl.ds(o, 16)]])
        vb = plsc.load_gather(b_v, [dl_v[pl.ds(o, 16)]])
        o_v[pl.ds(o, 16)] = va + vb + c_v[pl.ds(o, 16)]
        return _

    lax.fori_loop(0, EPW // 16, body, None)
    pltpu.sync_copy(o_v, out_hbm.at[pl.ds(base, EPW)])


def _lrelu(v):
    return jnp.where(v >= 0, v, 0.01 * v)


def _dot(a, b):
    return jax.lax.dot(a, b, preferred_element_type=jnp.float32)


def _tc1_body(x_ref, w1_ref, b1_ref, w2_ref, b2_ref, wc1_ref, degp_ref,
              hws1_ref, dis_ref):
    h = _lrelu(_dot(x_ref[...], w1_ref[...]) + b1_ref[...][None, :])
    h = _lrelu(_dot(h, w2_ref[...]) + b2_ref[...][None, :])
    deg = jnp.sum(degp_ref[...], axis=0) + 1.0
    dis = lax.rsqrt(deg)
    hws1_ref[pl.ds(0, N), :] = _dot(h, wc1_ref[...]) * dis[:, None]
    dis_ref[pl.ds(0, N)] = dis


_tc1 = pl.pallas_call(
    _tc1_body,
    out_shape=(jax.ShapeDtypeStruct((NP, H), jnp.float32),
               jax.ShapeDtypeStruct((NP,), jnp.float32)),
)


def _tc2_body(t1p_ref, hws1_ref, dis_ref, wc2_ref, bc1_ref, hws2_ref):
    dis = dis_ref[...]
    agg = (t1p_ref[0] + t1p_ref[1] + hws1_ref[...]) * dis[:, None] \
        + bc1_ref[...][None, :]
    h1 = _lrelu(agg)
    hws2_ref[...] = _dot(h1, wc2_ref[...]) * dis[:, None]


_tc2 = pl.pallas_call(
    _tc2_body,
    out_shape=jax.ShapeDtypeStruct((NP, H), jnp.float32),
)


def _tc3_body(t2p_ref, hws2_ref, dis_ref, bc2_ref, wpa_ref, wpb_ref,
              h2_ref, a_ref, bv_ref):
    dis = dis_ref[...]
    agg = (t2p_ref[0] + t2p_ref[1] + hws2_ref[...]) * dis[:, None] \
        + bc2_ref[...][None, :]
    h2 = _lrelu(agg)
    h2_ref[...] = h2[:N, :]
    a_ref[...] = jnp.sum(h2 * wpa_ref[...][None, :], axis=1)[:N]
    bv_ref[...] = jnp.sum(h2 * wpb_ref[...][None, :], axis=1)[:N]


_tc3 = pl.pallas_call(
    _tc3_body,
    out_shape=(jax.ShapeDtypeStruct((N, H), jnp.float32),
               jax.ShapeDtypeStruct((N,), jnp.float32),
               jax.ShapeDtypeStruct((N,), jnp.float32)),
)


def _tc4_body(ea_ref, wpe_ref, bp_ref, c_ref):
    i = pl.program_id(0)
    c_ref[pl.ds(i * _EB, _EB)] = \
        jnp.sum(ea_ref[...] * wpe_ref[...][:, None], axis=0) + bp_ref[0]


_EB = 32000

_tc4 = pl.pallas_call(
    _tc4_body,
    grid=(E // _EB,),
    in_specs=[
        pl.BlockSpec((H, _EB), lambda i: (0, i)),
        pl.BlockSpec((H,), lambda i: (0,)),
        pl.BlockSpec((1,), lambda i: (0,)),
    ],
    out_specs=pl.BlockSpec((E,), lambda i: (0,)),
    out_shape=jax.ShapeDtypeStruct((E,), jnp.float32),
)


def kernel(x, edge_index, edge_label_index, edge_attr,
           W1, b1, W2, b2, Wc1, bc1, Wc2, bc2, Wp, bp):
    src = edge_index[0]
    dst = edge_index[1]
    src2 = src.reshape(E // CH, CH)
    dst2 = dst.reshape(E // CH, CH)
    zn = jnp.zeros((N,), jnp.float32)
    znh = jnp.zeros((NP, H), jnp.float32)
    wp_flat = Wp.reshape(48)
    wpa, wpb, wpe = wp_flat[0:16], wp_flat[16:32], wp_flat[32:48]

    degp = _deg_kernel(dst, zn).reshape(NW, N)
    hws1, dis = _tc1(x, W1, b1, W2, b2, Wc1, degp)
    t1p = _seg_kernel(hws1, src2, dst2, znh)
    hws2 = _tc2(t1p, hws1, dis, Wc2, bc1)
    t2p = _seg_kernel(hws2, src2, dst2, znh)
    h2, a, bv = _tc3(t2p, hws2, dis, bc2, wpa, wpb)
    cvec = _tc4(edge_attr.T, wpe, bp)
    out = _score_kernel(a, bv, cvec, edge_label_index[0], edge_label_index[1])
    return (out, h2)


# speedup vs baseline: 2.0142x; 1.0072x over previous
import functools

import jax
import jax.numpy as jnp
from jax import lax
from jax.experimental import pallas as pl
from jax.experimental.pallas import tpu as pltpu
from jax.experimental.pallas import tpu_sc as plsc

N = 10000
E = 320000
H = 16
NC = 2
NS = 16
NW = NC * NS
EPW = E // NW
CH = 125
NCH = EPW // CH
NP = 10240
NPW = NP // NS

_mesh = plsc.VectorSubcoreMesh(core_axis_name="c", subcore_axis_name="s")


def _take16(v, ix):
    return lax.gather(
        v, ix[:, None],
        dimension_numbers=lax.GatherDimensionNumbers(
            offset_dims=(), collapsed_slice_dims=(0,), start_index_map=(0,)),
        slice_sizes=(1,),
        mode=lax.GatherScatterMode.PROMISE_IN_BOUNDS)


@functools.partial(
    pl.kernel,
    out_type=jax.ShapeDtypeStruct((NW * N,), jnp.float32),
    mesh=_mesh,
    scratch_types=[
        pltpu.VMEM((N,), jnp.float32),
        pltpu.VMEM((EPW,), jnp.int32),
    ],
    compiler_params=pltpu.CompilerParams(needs_layout_passes=False,
                                         use_tc_tiling_on_sc=False),
)
def _deg_kernel(dst_hbm, zn_hbm, out_hbm, deg_v, idx_v):
    c = lax.axis_index("c")
    s = lax.axis_index("s")
    wid = s * NC + c
    pltpu.sync_copy(zn_hbm, deg_v)
    pltpu.sync_copy(dst_hbm.at[pl.ds(wid * EPW, EPW)], idx_v)
    lanes = lax.iota(jnp.int32, 16)
    prev_ix = jnp.maximum(lanes - 1, 0)
    next_ix = jnp.minimum(lanes + 1, 15)

    def body(i, _):
        idx = idx_v[pl.ds(i * 16, 16)]
        sk = jnp.sort(idx)
        prev = _take16(sk, prev_ix)
        nxt = _take16(sk, next_ix)
        start = (sk != prev) | (lanes == 0)
        islast = (sk != nxt) | (lanes == 15)
        runstart = plsc.cummax(jnp.where(start, lanes, 0))
        cnt = (lanes - runstart + 1).astype(jnp.float32)
        plsc.addupdate_scatter(deg_v, [sk], cnt, mask=islast)
        return _

    lax.fori_loop(0, EPW // 16, body, None)
    pltpu.sync_copy(deg_v, out_hbm.at[pl.ds(wid * N, N)])


@functools.partial(
    pl.kernel,
    out_type=jax.ShapeDtypeStruct((NC, NP, H), jnp.float32),
    mesh=_mesh,
    scratch_types=[
        pltpu.VMEM_SHARED((NP, H), jnp.float32),
        pltpu.VMEM((NCH, CH), jnp.int32),
        pltpu.VMEM((NCH, CH), jnp.int32),
        pltpu.VMEM((2, CH, H), jnp.float32),
        pltpu.SemaphoreType.DMA,
        pltpu.SemaphoreType.DMA,
    ],
    compiler_params=pltpu.CompilerParams(use_tc_tiling_on_sc=False),
)
def _seg_kernel(hws_hbm, src2_hbm, dst2_hbm, znh_hbm, out_hbm,
                acc_sh, sidx_v, didx_v, rows_v, sem0, sem1):
    c = lax.axis_index("c")
    s = lax.axis_index("s")
    wid = s * NC + c

    pltpu.sync_copy(src2_hbm.at[pl.ds(wid * NCH, NCH)], sidx_v)
    pltpu.sync_copy(dst2_hbm.at[pl.ds(wid * NCH, NCH)], didx_v)
    pltpu.sync_copy(znh_hbm.at[pl.ds(s * NPW, NPW)],
                    acc_sh.at[pl.ds(s * NPW, NPW)])

    def _gather(j, buf, sem):
        return pltpu.async_copy(hws_hbm.at[sidx_v.at[j]], rows_v.at[buf], sem)

    def _gwait(j, buf, sem):
        pltpu.make_async_copy(hws_hbm.at[sidx_v.at[j]], rows_v.at[buf], sem).wait()

    def _scatter(j, buf):
        pltpu.sync_copy(rows_v.at[buf], acc_sh.at[didx_v.at[j]], add=True)

    _gather(0, 0, sem0)
    plsc.subcore_barrier()

    def body(k, _):
        j0 = 2 * k
        j1 = 2 * k + 1
        _gather(j1, 1, sem1)
        _gwait(j0, 0, sem0)
        _scatter(j0, 0)
        _gather(j1 + 1, 0, sem0)
        _gwait(j1, 1, sem1)
        _scatter(j1, 1)
        return _

    lax.fori_loop(0, NCH // 2 - 1, body, None)
    _gather(NCH - 1, 1, sem1)
    _gwait(NCH - 2, 0, sem0)
    _scatter(NCH - 2, 0)
    _gwait(NCH - 1, 1, sem1)
    _scatter(NCH - 1, 1)
    plsc.subcore_barrier()
    pltpu.sync_copy(acc_sh.at[pl.ds(s * NPW, NPW)],
                    out_hbm.at[c].at[pl.ds(s * NPW, NPW)])


@functools.partial(
    pl.kernel,
    out_type=jax.ShapeDtypeStruct((E,), jnp.float32),
    mesh=_mesh,
    scratch_types=[
        pltpu.VMEM((N,), jnp.float32),
        pltpu.VMEM((N,), jnp.float32),
        pltpu.VMEM((EPW,), jnp.int32),
        pltpu.VMEM((EPW,), jnp.int32),
        pltpu.VMEM((EPW,), jnp.float32),
        pltpu.VMEM((EPW,), jnp.float32),
    ],
    compiler_params=pltpu.CompilerParams(needs_layout_passes=False,
                                         use_tc_tiling_on_sc=False),
)
def _score_kernel(a_hbm, b_hbm, cc_hbm, sl_hbm, dl_hbm, out_hbm,
                  a_v, b_v, sl_v, dl_v, c_v, o_v):
    c = lax.axis_index("c")
    s = lax.axis_index("s")
    wid = s * NC + c
    base = wid * EPW
    pltpu.sync_copy(a_hbm, a_v)
    pltpu.sync_copy(b_hbm, b_v)
    pltpu.sync_copy(sl_hbm.at[pl.ds(base, EPW)], sl_v)
    pltpu.sync_copy(dl_hbm.at[pl.ds(base, EPW)], dl_v)
    pltpu.sync_copy(cc_hbm.at[pl.ds(base, EPW)], c_v)

    def body(i, _):
        o = i * 16
        va = plsc.load_gather(a_v, [sl_v[pl.ds(o, 16)]])
        vb = plsc.load_gather(b_v, [dl_v[pl.ds(o, 16)]])
        o_v[pl.ds(o, 16)] = va + vb + c_v[pl.ds(o, 16)]
        return _

    lax.fori_loop(0, EPW // 16, body, None)
    pltpu.sync_copy(o_v, out_hbm.at[pl.ds(base, EPW)])


def _lrelu(v):
    return jnp.where(v >= 0, v, 0.01 * v)


def _dot(a, b):
    return jax.lax.dot(a, b, preferred_element_type=jnp.float32)


def _tc1a_body(x_ref, w1_ref, b1_ref, w2_ref, b2_ref, wc1_ref, hw1_ref):
    h = _lrelu(_dot(x_ref[...], w1_ref[...]) + b1_ref[...][None, :])
    h = _lrelu(_dot(h, w2_ref[...]) + b2_ref[...][None, :])
    hw1_ref[...] = _dot(h, wc1_ref[...])


_tc1a = pl.pallas_call(
    _tc1a_body,
    out_shape=jax.ShapeDtypeStruct((N, H), jnp.float32),
)


def _tc1b_body(hw1_ref, degp_ref, hws1_ref, dis_ref):
    deg = jnp.sum(degp_ref[...], axis=0) + 1.0
    dis = lax.rsqrt(deg)
    hws1_ref[pl.ds(0, N), :] = hw1_ref[...] * dis[:, None]
    dis_ref[pl.ds(0, N)] = dis


_tc1b = pl.pallas_call(
    _tc1b_body,
    out_shape=(jax.ShapeDtypeStruct((NP, H), jnp.float32),
               jax.ShapeDtypeStruct((NP,), jnp.float32)),
)


def _tc2_body(t1p_ref, hws1_ref, dis_ref, wc2_ref, bc1_ref, hws2_ref):
    dis = dis_ref[...]
    agg = (t1p_ref[0] + t1p_ref[1] + hws1_ref[...]) * dis[:, None] \
        + bc1_ref[...][None, :]
    h1 = _lrelu(agg)
    hws2_ref[...] = _dot(h1, wc2_ref[...]) * dis[:, None]


_tc2 = pl.pallas_call(
    _tc2_body,
    out_shape=jax.ShapeDtypeStruct((NP, H), jnp.float32),
)


def _tc3_body(t2p_ref, hws2_ref, dis_ref, bc2_ref, wpa_ref, wpb_ref,
              h2_ref, a_ref, bv_ref):
    dis = dis_ref[...]
    agg = (t2p_ref[0] + t2p_ref[1] + hws2_ref[...]) * dis[:, None] \
        + bc2_ref[...][None, :]
    h2 = _lrelu(agg)
    h2_ref[...] = h2[:N, :]
    a_ref[...] = jnp.sum(h2 * wpa_ref[...][None, :], axis=1)[:N]
    bv_ref[...] = jnp.sum(h2 * wpb_ref[...][None, :], axis=1)[:N]


_tc3 = pl.pallas_call(
    _tc3_body,
    out_shape=(jax.ShapeDtypeStruct((N, H), jnp.float32),
               jax.ShapeDtypeStruct((N,), jnp.float32),
               jax.ShapeDtypeStruct((N,), jnp.float32)),
)


def _tc4_body(ea_ref, wpe_ref, bp_ref, c_ref):
    i = pl.program_id(0)
    c_ref[pl.ds(i * _EB, _EB)] = \
        jnp.sum(ea_ref[...] * wpe_ref[...][:, None], axis=0) + bp_ref[0]


_EB = 32000

_tc4 = pl.pallas_call(
    _tc4_body,
    grid=(E // _EB,),
    in_specs=[
        pl.BlockSpec((H, _EB), lambda i: (0, i)),
        pl.BlockSpec((H,), lambda i: (0,)),
        pl.BlockSpec((1,), lambda i: (0,)),
    ],
    out_specs=pl.BlockSpec((E,), lambda i: (0,)),
    out_shape=jax.ShapeDtypeStruct((E,), jnp.float32),
)


def kernel(x, edge_index, edge_label_index, edge_attr,
           W1, b1, W2, b2, Wc1, bc1, Wc2, bc2, Wp, bp):
    src = edge_index[0]
    dst = edge_index[1]
    src2 = src.reshape(E // CH, CH)
    dst2 = dst.reshape(E // CH, CH)
    zn = jnp.zeros((N,), jnp.float32)
    znh = jnp.zeros((NP, H), jnp.float32)
    wp_flat = Wp.reshape(48)
    wpa, wpb, wpe = wp_flat[0:16], wp_flat[16:32], wp_flat[32:48]

    degp = _deg_kernel(dst, zn).reshape(NW, N)
    hw1 = _tc1a(x, W1, b1, W2, b2, Wc1)
    hws1, dis = _tc1b(hw1, degp)
    t1p = _seg_kernel(hws1, src2, dst2, znh)
    hws2 = _tc2(t1p, hws1, dis, Wc2, bc1)
    t2p = _seg_kernel(hws2, src2, dst2, znh)
    h2, a, bv = _tc3(t2p, hws2, dis, bc2, wpa, wpb)
    cvec = _tc4(edge_attr.T, wpe, bp)
    out = _score_kernel(a, bv, cvec, edge_label_index[0], edge_label_index[1])
    return (out, h2)
